# baseline (speedup 1.0000x reference)
import jax
import jax.numpy as jnp
from jax.experimental import pallas as pl
from jax.experimental.pallas import tpu as pltpu

_CHUNKS = 8


def _copy_body(u_hbm, i_hbm, uo_hbm, io_hbm, sem_u, sem_i):
    n = u_hbm.shape[0]
    c = n // _CHUNKS
    copies = []
    for k in range(_CHUNKS):
        sl = pl.ds(k * c, c)
        cu = pltpu.make_async_copy(u_hbm.at[sl], uo_hbm.at[sl], sem_u.at[k])
        ci = pltpu.make_async_copy(i_hbm.at[sl], io_hbm.at[sl], sem_i.at[k])
        cu.start()
        ci.start()
        copies.append((cu, ci))
    for cu, ci in copies:
        cu.wait()
        ci.wait()


def kernel(user_emb, item_emb):
    n, d = user_emb.shape
    u1 = user_emb.reshape(n * d)
    i1 = item_emb.reshape(n * d)
    out = pl.pallas_call(
        _copy_body,
        in_specs=[
            pl.BlockSpec(memory_space=pltpu.MemorySpace.HBM),
            pl.BlockSpec(memory_space=pltpu.MemorySpace.HBM),
        ],
        out_specs=[
            pl.BlockSpec(memory_space=pltpu.MemorySpace.HBM),
            pl.BlockSpec(memory_space=pltpu.MemorySpace.HBM),
        ],
        out_shape=[
            jax.ShapeDtypeStruct((n * d,), user_emb.dtype),
            jax.ShapeDtypeStruct((n * d,), item_emb.dtype),
        ],
        scratch_shapes=[
            pltpu.SemaphoreType.DMA((_CHUNKS,)),
            pltpu.SemaphoreType.DMA((_CHUNKS,)),
        ],
    )(u1, i1)
    return (out[0].reshape(n, d), out[1].reshape(n, d))


# speedup vs baseline: 3.2943x; 3.2943x over previous
import functools

import jax
import jax.numpy as jnp
from jax import lax
from jax.experimental import pallas as pl
from jax.experimental.pallas import tpu as pltpu
from jax.experimental.pallas import tpu_sc as plsc

_NUM_CORES = 2
_NUM_SUBCORES = 16
_NUM_WORKERS = _NUM_CORES * _NUM_SUBCORES
_MAX_CHUNK_BYTES = 240 * 1024


def _pick_chunks(rows_per_worker):
    for cnt in range(1, rows_per_worker + 1):
        if rows_per_worker % cnt:
            continue
        ch = rows_per_worker // cnt
        if ch % 8:
            continue
        if ch * 16 * 4 <= _MAX_CHUNK_BYTES:
            return cnt, ch
    raise ValueError("no valid chunking")


def _make_copy_kernel(n, d, dtype):
    assert n % 8 == 0 and d == 16
    rows_per_worker = (n // (8 * _NUM_WORKERS)) * 8
    tail_rows = n - rows_per_worker * _NUM_WORKERS
    tail_workers = tail_rows // 8
    chunks_per_table, chunk_rows = _pick_chunks(rows_per_worker)
    mesh = plsc.VectorSubcoreMesh(
        core_axis_name="c", subcore_axis_name="s",
        num_cores=_NUM_CORES, num_subcores=_NUM_SUBCORES,
    )

    @functools.partial(
        pl.kernel,
        out_type=(
            jax.ShapeDtypeStruct((n, d), dtype),
            jax.ShapeDtypeStruct((n, d), dtype),
        ),
        mesh=mesh,
        scratch_types=(
            pltpu.VMEM((chunk_rows, d), dtype),
            pltpu.VMEM((chunk_rows, d), dtype),
            pltpu.SemaphoreType.DMA((2,)),
            pltpu.SemaphoreType.DMA((2,)),
        ),
        compiler_params=pltpu.CompilerParams(use_tc_tiling_on_sc=False),
    )
    def copy_kernel(u_hbm, i_hbm, uo_hbm, io_hbm, b0, b1, sem_in, sem_out):
        wid = lax.axis_index("s") * _NUM_CORES + lax.axis_index("c")
        base = pl.multiple_of(wid * rows_per_worker, 8)
        bufs = (b0, b1)
        tasks = []
        for src, dst in ((u_hbm, uo_hbm), (i_hbm, io_hbm)):
            for j in range(chunks_per_table):
                tasks.append((src, dst, j * chunk_rows))
        n_t = len(tasks)

        def start_in(j):
            src, _, off = tasks[j]
            c = pltpu.make_async_copy(
                src.at[pl.ds(base + off, chunk_rows), :], bufs[j % 2],
                sem_in.at[j % 2])
            c.start()
            return c

        def start_out(j):
            _, dst, off = tasks[j]
            c = pltpu.make_async_copy(
                bufs[j % 2], dst.at[pl.ds(base + off, chunk_rows), :],
                sem_out.at[j % 2])
            c.start()
            return c

        in_d = [None] * n_t
        out_d = [None] * n_t
        in_d[0] = start_in(0)
        for j in range(n_t):
            if j + 1 < n_t:
                if j >= 1:
                    out_d[j - 1].wait()
                in_d[j + 1] = start_in(j + 1)
            in_d[j].wait()
            out_d[j] = start_out(j)
        if n_t >= 2:
            out_d[n_t - 2].wait()
        out_d[n_t - 1].wait()

        if tail_rows:
            tail_base = rows_per_worker * _NUM_WORKERS

            @pl.when(wid < tail_workers)
            def _():
                toff = pl.multiple_of(tail_base + wid * 8, 8)
                for src, dst in ((u_hbm, uo_hbm), (i_hbm, io_hbm)):
                    cin = pltpu.make_async_copy(
                        src.at[pl.ds(toff, 8), :],
                        b0.at[pl.ds(0, 8), :], sem_in.at[0])
                    cin.start()
                    cin.wait()
                    cout = pltpu.make_async_copy(
                        b0.at[pl.ds(0, 8), :],
                        dst.at[pl.ds(toff, 8), :], sem_out.at[0])
                    cout.start()
                    cout.wait()

    return copy_kernel


def kernel(user_emb, item_emb):
    n, d = user_emb.shape
    out = _make_copy_kernel(n, d, user_emb.dtype)(user_emb, item_emb)
    return (out[0], out[1])


# speedup vs baseline: 48.6003x; 14.7527x over previous
import functools

import jax
import jax.numpy as jnp
from jax import lax
from jax.experimental import pallas as pl
from jax.experimental.pallas import tpu as pltpu
from jax.experimental.pallas import tpu_sc as plsc

_NUM_CORES = 2
_NUM_SUBCORES = 16
_NUM_WORKERS = _NUM_CORES * _NUM_SUBCORES
_LANE = 128
_CHUNK_TILES = 30


def _make_copy_kernel(n, d, dtype):
    full_tiles = n // _LANE
    tail_cols = n % _LANE
    per_worker_tiles = full_tiles // _NUM_WORKERS
    rem_tiles = full_tiles % _NUM_WORKERS
    main_chunks, last = divmod(per_worker_tiles, _CHUNK_TILES)
    chunk_tiles = [_CHUNK_TILES] * main_chunks + ([last] if last else [])
    mesh = plsc.VectorSubcoreMesh(
        core_axis_name="c", subcore_axis_name="s",
        num_cores=_NUM_CORES, num_subcores=_NUM_SUBCORES,
    )
    buf_cols = _CHUNK_TILES * _LANE

    @functools.partial(
        pl.kernel,
        out_type=(
            jax.ShapeDtypeStruct((d, n), dtype),
            jax.ShapeDtypeStruct((d, n), dtype),
        ),
        mesh=mesh,
        scratch_types=(
            pltpu.VMEM((d, buf_cols), dtype),
            pltpu.VMEM((d, buf_cols), dtype),
            pltpu.SemaphoreType.DMA((2,)),
            pltpu.SemaphoreType.DMA((2,)),
        ),
    )
    def copy_kernel(u_hbm, i_hbm, uo_hbm, io_hbm, b0, b1, sem_in, sem_out):
        wid = lax.axis_index("s") * _NUM_CORES + lax.axis_index("c")
        base = pl.multiple_of(wid * (per_worker_tiles * _LANE), _LANE)
        bufs = (b0, b1)
        tasks = []
        for src, dst in ((u_hbm, uo_hbm), (i_hbm, io_hbm)):
            off = 0
            for t in chunk_tiles:
                tasks.append((src, dst, off, t * _LANE))
                off += t * _LANE
        n_t = len(tasks)

        def start_in(j):
            src, _, off, cols = tasks[j]
            buf = bufs[j % 2]
            c = pltpu.make_async_copy(
                src.at[:, pl.ds(base + off, cols)],
                buf.at[:, pl.ds(0, cols)], sem_in.at[j % 2])
            c.start()
            return c

        def start_out(j):
            _, dst, off, cols = tasks[j]
            buf = bufs[j % 2]
            c = pltpu.make_async_copy(
                buf.at[:, pl.ds(0, cols)],
                dst.at[:, pl.ds(base + off, cols)], sem_out.at[j % 2])
            c.start()
            return c

        in_d = [None] * n_t
        out_d = [None] * n_t
        in_d[0] = start_in(0)
        for j in range(n_t):
            if j + 1 < n_t:
                if j >= 1:
                    out_d[j - 1].wait()
                in_d[j + 1] = start_in(j + 1)
            in_d[j].wait()
            out_d[j] = start_out(j)
        if n_t >= 2:
            out_d[n_t - 2].wait()
        out_d[n_t - 1].wait()

        rem_base = per_worker_tiles * _NUM_WORKERS * _LANE
        if rem_tiles:
            @pl.when(wid < rem_tiles)
            def _():
                toff = pl.multiple_of(rem_base + wid * _LANE, _LANE)
                for src, dst in ((u_hbm, uo_hbm), (i_hbm, io_hbm)):
                    cin = pltpu.make_async_copy(
                        src.at[:, pl.ds(toff, _LANE)],
                        b0.at[:, pl.ds(0, _LANE)], sem_in.at[0])
                    cin.start()
                    cin.wait()
                    cout = pltpu.make_async_copy(
                        b0.at[:, pl.ds(0, _LANE)],
                        dst.at[:, pl.ds(toff, _LANE)], sem_out.at[0])
                    cout.start()
                    cout.wait()

    return copy_kernel


def kernel(user_emb, item_emb):
    n, d = user_emb.shape
    ut, it = user_emb.T, item_emb.T
    out = _make_copy_kernel(n, d, user_emb.dtype)(ut, it)
    n_main = (n // _LANE) * _LANE
    if n_main != n:
        ou = lax.dynamic_update_slice(
            out[0], lax.slice(ut, (0, n_main), (d, n)), (0, n_main))
        oi = lax.dynamic_update_slice(
            out[1], lax.slice(it, (0, n_main), (d, n)), (0, n_main))
    else:
        ou, oi = out
    return (ou.T, oi.T)


# speedup vs baseline: 49.2702x; 1.0138x over previous
import functools
import math

import jax
import jax.numpy as jnp
from jax import lax
from jax.experimental import pallas as pl
from jax.experimental.pallas import tpu as pltpu
from jax.experimental.pallas import tpu_sc as plsc

_NUM_CORES = 2
_NUM_SUBCORES = 16
_NUM_WORKERS = _NUM_CORES * _NUM_SUBCORES
_LANE = 128
_CHUNK_TILES = 30
_TC_BLOCK_COLS = 16384


def _make_sc_copy(n, d, dtype):
    full_tiles = n // _LANE
    per_worker_tiles = full_tiles // _NUM_WORKERS
    rem_tiles = full_tiles % _NUM_WORKERS
    main_chunks, last = divmod(per_worker_tiles, _CHUNK_TILES)
    chunk_tiles = [_CHUNK_TILES] * main_chunks + ([last] if last else [])
    mesh = plsc.VectorSubcoreMesh(
        core_axis_name="c", subcore_axis_name="s",
        num_cores=_NUM_CORES, num_subcores=_NUM_SUBCORES,
    )
    buf_cols = _CHUNK_TILES * _LANE

    @functools.partial(
        pl.kernel,
        out_type=jax.ShapeDtypeStruct((d, n), dtype),
        mesh=mesh,
        scratch_types=(
            pltpu.VMEM((d, buf_cols), dtype),
            pltpu.VMEM((d, buf_cols), dtype),
            pltpu.SemaphoreType.DMA((2,)),
            pltpu.SemaphoreType.DMA((2,)),
        ),
    )
    def sc_copy(x_hbm, o_hbm, b0, b1, sem_in, sem_out):
        wid = lax.axis_index("s") * _NUM_CORES + lax.axis_index("c")
        base = pl.multiple_of(wid * (per_worker_tiles * _LANE), _LANE)
        bufs = (b0, b1)
        tasks = []
        off = 0
        for t in chunk_tiles:
            tasks.append((off, t * _LANE))
            off += t * _LANE
        n_t = len(tasks)

        def start_in(j):
            off, cols = tasks[j]
            c = pltpu.make_async_copy(
                x_hbm.at[:, pl.ds(base + off, cols)],
                bufs[j % 2].at[:, pl.ds(0, cols)], sem_in.at[j % 2])
            c.start()
            return c

        def start_out(j):
            off, cols = tasks[j]
            c = pltpu.make_async_copy(
                bufs[j % 2].at[:, pl.ds(0, cols)],
                o_hbm.at[:, pl.ds(base + off, cols)], sem_out.at[j % 2])
            c.start()
            return c

        in_d = [None] * n_t
        out_d = [None] * n_t
        in_d[0] = start_in(0)
        for j in range(n_t):
            if j + 1 < n_t:
                if j >= 1:
                    out_d[j - 1].wait()
                in_d[j + 1] = start_in(j + 1)
            in_d[j].wait()
            out_d[j] = start_out(j)
        if n_t >= 2:
            out_d[n_t - 2].wait()
        out_d[n_t - 1].wait()

        if rem_tiles:
            rem_base = per_worker_tiles * _NUM_WORKERS * _LANE

            @pl.when(wid < rem_tiles)
            def _():
                toff = pl.multiple_of(rem_base + wid * _LANE, _LANE)
                cin = pltpu.make_async_copy(
                    x_hbm.at[:, pl.ds(toff, _LANE)],
                    b0.at[:, pl.ds(0, _LANE)], sem_in.at[0])
                cin.start()
                cin.wait()
                cout = pltpu.make_async_copy(
                    b0.at[:, pl.ds(0, _LANE)],
                    o_hbm.at[:, pl.ds(toff, _LANE)], sem_out.at[0])
                cout.start()
                cout.wait()

    return sc_copy


def _tc_body(x_ref, o_ref):
    o_ref[...] = x_ref[...]


def _tc_copy(x):
    d, n = x.shape
    grid = math.ceil(n / _TC_BLOCK_COLS)
    spec = pl.BlockSpec((d, _TC_BLOCK_COLS), lambda i: (0, i))
    return pl.pallas_call(
        _tc_body,
        grid=(grid,),
        in_specs=[spec],
        out_specs=spec,
        out_shape=jax.ShapeDtypeStruct((d, n), x.dtype),
    )(x)


def kernel(user_emb, item_emb):
    n, d = user_emb.shape
    ut, it = user_emb.T, item_emb.T
    ou = _make_sc_copy(n, d, user_emb.dtype)(ut)
    oi = _tc_copy(it)
    n_main = (n // _LANE) * _LANE
    if n_main != n:
        ou = lax.dynamic_update_slice(
            ou, lax.slice(ut, (0, n_main), (d, n)), (0, n_main))
    return (ou.T, oi.T)


# speedup vs baseline: 49.4153x; 1.0029x over previous
import functools
import math

import jax
import jax.numpy as jnp
from jax import lax
from jax.experimental import pallas as pl
from jax.experimental.pallas import tpu as pltpu
from jax.experimental.pallas import tpu_sc as plsc

_NUM_CORES = 2
_NUM_SUBCORES = 16
_NUM_WORKERS = _NUM_CORES * _NUM_SUBCORES
_LANE = 128
_CHUNK_TILES = 30
_TC_BLOCK_COLS = 16384


def _make_sc_copy(n, d, dtype):
    full_tiles = n // _LANE
    per_worker_tiles = full_tiles // _NUM_WORKERS
    rem_tiles = full_tiles % _NUM_WORKERS
    main_chunks, last = divmod(per_worker_tiles, _CHUNK_TILES)
    chunk_tiles = [_CHUNK_TILES] * main_chunks + ([last] if last else [])
    mesh = plsc.VectorSubcoreMesh(
        core_axis_name="c", subcore_axis_name="s",
        num_cores=_NUM_CORES, num_subcores=_NUM_SUBCORES,
    )
    buf_cols = _CHUNK_TILES * _LANE

    @functools.partial(
        pl.kernel,
        out_type=jax.ShapeDtypeStruct((d, n), dtype),
        mesh=mesh,
        scratch_types=(
            pltpu.VMEM((d, buf_cols), dtype),
            pltpu.VMEM((d, buf_cols), dtype),
            pltpu.SemaphoreType.DMA((2,)),
            pltpu.SemaphoreType.DMA((2,)),
        ),
    )
    def sc_copy(x_hbm, o_hbm, b0, b1, sem_in, sem_out):
        wid = lax.axis_index("s") * _NUM_CORES + lax.axis_index("c")
        base = pl.multiple_of(wid * (per_worker_tiles * _LANE), _LANE)
        bufs = (b0, b1)
        tasks = []
        off = 0
        for t in chunk_tiles:
            tasks.append((off, t * _LANE))
            off += t * _LANE
        n_t = len(tasks)

        def start_in(j):
            off, cols = tasks[j]
            c = pltpu.make_async_copy(
                x_hbm.at[:, pl.ds(base + off, cols)],
                bufs[j % 2].at[:, pl.ds(0, cols)], sem_in.at[j % 2])
            c.start()
            return c

        def start_out(j):
            off, cols = tasks[j]
            c = pltpu.make_async_copy(
                bufs[j % 2].at[:, pl.ds(0, cols)],
                o_hbm.at[:, pl.ds(base + off, cols)], sem_out.at[j % 2])
            c.start()
            return c

        in_d = [None] * n_t
        out_d = [None] * n_t
        in_d[0] = start_in(0)
        for j in range(n_t):
            if j + 1 < n_t:
                if j >= 1:
                    out_d[j - 1].wait()
                in_d[j + 1] = start_in(j + 1)
            in_d[j].wait()
            out_d[j] = start_out(j)
        if n_t >= 2:
            out_d[n_t - 2].wait()
        out_d[n_t - 1].wait()

        if rem_tiles:
            rem_base = per_worker_tiles * _NUM_WORKERS * _LANE

            @pl.when(wid < rem_tiles)
            def _():
                toff = pl.multiple_of(rem_base + wid * _LANE, _LANE)
                cin = pltpu.make_async_copy(
                    x_hbm.at[:, pl.ds(toff, _LANE)],
                    b0.at[:, pl.ds(0, _LANE)], sem_in.at[0])
                cin.start()
                cin.wait()
                cout = pltpu.make_async_copy(
                    b0.at[:, pl.ds(0, _LANE)],
                    o_hbm.at[:, pl.ds(toff, _LANE)], sem_out.at[0])
                cout.start()
                cout.wait()

    return sc_copy


def _tc_body(x_ref, o_ref):
    o_ref[...] = x_ref[...]


def _tc_copy(x):
    d, n = x.shape
    grid = math.ceil(n / _TC_BLOCK_COLS)
    spec = pl.BlockSpec((d, _TC_BLOCK_COLS), lambda i: (0, i))
    return pl.pallas_call(
        _tc_body,
        grid=(grid,),
        in_specs=[spec],
        out_specs=spec,
        out_shape=jax.ShapeDtypeStruct((d, n), x.dtype),
        compiler_params=pltpu.CompilerParams(skip_device_barrier=True),
    )(x)


def kernel(user_emb, item_emb):
    n, d = user_emb.shape
    ut, it = user_emb.T, item_emb.T
    ou = _make_sc_copy(n, d, user_emb.dtype)(ut)
    oi = _tc_copy(it)
    n_main = (n // _LANE) * _LANE
    if n_main != n:
        ou = lax.dynamic_update_slice(
            ou, lax.slice(ut, (0, n_main), (d, n)), (0, n_main))
    return (ou.T, oi.T)
